# Initial kernel scaffold; baseline (speedup 1.0000x reference)
import jax
import jax.numpy as jnp
from jax.experimental import pallas as pl


def kernel(logits, features, W, b):
    raise NotImplementedError("write your pallas kernel here")


# speedup vs baseline: 2.2678x; 2.2678x over previous
import functools
import jax
import jax.numpy as jnp
from jax.experimental import pallas as pl
from jax.experimental.pallas import tpu as pltpu

_R = 256
_C = 4096
_ROWS = 1024
_COLS = 100000


def _topk_body(w_ref, b_ref, x_ref, o_ref, t_ref, *, nj, blk_c, n_cols):
    j = pl.program_id(1)

    @pl.when(j == 0)
    def _init():
        t_ref[...] = jnp.full_like(t_ref, -jnp.inf)

    base = j * blk_c
    lane = jax.lax.broadcasted_iota(jnp.int32, (t_ref.shape[1], 128), 1)

    T = [t_ref[k] for k in range(5)]
    for c in range(blk_c // 128):
        v = x_ref[:, c * 128:(c + 1) * 128]
        v = jnp.where(base + c * 128 + lane < n_cols, v, -jnp.inf)
        for k in range(5):
            hi = jnp.maximum(T[k], v)
            v = jnp.minimum(T[k], v)
            T[k] = hi
    for k in range(5):
        t_ref[k] = T[k]

    @pl.when(j == nj - 1)
    def _final():
        Tf = [t_ref[k] for k in range(5)]
        z = b_ref[0]
        for k in range(5):
            m = jnp.max(Tf[0], axis=1, keepdims=True)
            eq = Tf[0] == m
            cand = jnp.where(eq, lane, 1 << 20)
            jmin = jnp.min(cand, axis=1, keepdims=True)
            hit = lane == jmin
            Tf[0] = jnp.where(hit, Tf[1], Tf[0])
            Tf[1] = jnp.where(hit, Tf[2], Tf[1])
            Tf[2] = jnp.where(hit, Tf[3], Tf[2])
            Tf[3] = jnp.where(hit, Tf[4], Tf[3])
            Tf[4] = jnp.where(hit, -jnp.inf, Tf[4])
            z = z + w_ref[0, k] * m
        o_ref[...] = jax.nn.sigmoid(z)


def kernel(logits, features, W, b):
    del features
    rows, n_cols = logits.shape
    nj = -(-n_cols // _C)
    body = functools.partial(_topk_body, nj=nj, blk_c=_C, n_cols=n_cols)
    out = pl.pallas_call(
        body,
        grid=(rows // _R, nj),
        in_specs=[
            pl.BlockSpec(memory_space=pltpu.SMEM),
            pl.BlockSpec(memory_space=pltpu.SMEM),
            pl.BlockSpec((_R, _C), lambda i, j: (i, j)),
        ],
        out_specs=pl.BlockSpec((_R, 1), lambda i, j: (i, 0)),
        out_shape=jax.ShapeDtypeStruct((rows, 1), jnp.float32),
        scratch_shapes=[pltpu.VMEM((5, _R, 128), jnp.float32)],
        compiler_params=pltpu.CompilerParams(
            dimension_semantics=("parallel", "arbitrary"),
        ),
    )(W, b, logits)
    return out
